# baseline (speedup 1.0000x reference)
import functools

import jax
import jax.numpy as jnp
from jax import lax
from jax.experimental import pallas as pl
from jax.experimental.pallas import tpu as pltpu
from jax.experimental.pallas import tpu_sc as plsc

B = 64
N_TOK = 196
D = 768
POOL = 512
TOPK = 4
L = 5
DUAL = 2
ROW = 20 * 12 * 64
N_ROWS_OUT = L * B * DUAL * TOPK
N_ROWS_TABLE = L * DUAL * POOL

B_BLK = 8


def _sim_topk_body(x_ref, pk_ref, sim_ref, idx_ref):
    x_mean = jnp.mean(x_ref[...], axis=1)
    x_norm = x_mean * lax.rsqrt(
        jnp.maximum(jnp.sum(x_mean * x_mean, axis=-1, keepdims=True), 1e-12))
    pk = pk_ref[...]
    pk_norm = pk * lax.rsqrt(
        jnp.maximum(jnp.sum(pk * pk, axis=-1, keepdims=True), 1e-12))
    sim = jnp.dot(x_norm, pk_norm.T,
                  preferred_element_type=jnp.float32)
    sim_ref[...] = sim

    iota = lax.broadcasted_iota(jnp.int32, (B_BLK, POOL), 1)
    cur = sim
    cols = []
    for _ in range(TOPK):
        m = jnp.max(cur, axis=1, keepdims=True)
        j = jnp.min(jnp.where(cur == m, iota, POOL), axis=1)
        cols.append(j[:, None])
        cur = jnp.where(iota == j[:, None], jnp.float32(-jnp.inf), cur)
    idx_ref[...] = jnp.concatenate(cols, axis=1)


def _sim_topk(x_embed, prompt_key):
    return pl.pallas_call(
        _sim_topk_body,
        grid=(B // B_BLK,),
        in_specs=[
            pl.BlockSpec((B_BLK, N_TOK, D), lambda i: (i, 0, 0)),
            pl.BlockSpec((POOL, D), lambda i: (0, 0)),
        ],
        out_specs=[
            pl.BlockSpec((B_BLK, POOL), lambda i: (i, 0)),
            pl.BlockSpec((B_BLK, TOPK), lambda i: (i, 0)),
        ],
        out_shape=[
            jax.ShapeDtypeStruct((B, POOL), jnp.float32),
            jax.ShapeDtypeStruct((B, TOPK), jnp.int32),
        ],
    )(x_embed, prompt_key)


_NW = 32
_RPW = N_ROWS_OUT // _NW
_CHUNK = 8
_NCHUNK = _RPW // _CHUNK


@functools.cache
def _sc_gather_fn():
    @functools.partial(
        pl.kernel,
        out_type=jax.ShapeDtypeStruct((N_ROWS_OUT, ROW), jnp.float32),
        mesh=plsc.VectorSubcoreMesh(core_axis_name="c", subcore_axis_name="s"),
        scratch_types=[
            pltpu.VMEM((_NCHUNK, _CHUNK), jnp.int32),
            pltpu.VMEM((_CHUNK, ROW), jnp.float32),
            pltpu.SemaphoreType.DMA,
        ],
    )
    def _sc_gather(table_hbm, src_hbm, out_hbm, idx_v, buf, sem):
        wid = lax.axis_index("s") * 2 + lax.axis_index("c")
        pltpu.sync_copy(src_hbm.at[wid], idx_v)
        base = wid * _RPW
        for g in range(_NCHUNK):
            pltpu.async_copy(table_hbm.at[idx_v.at[g]], buf, sem).wait()
            pltpu.sync_copy(buf, out_hbm.at[pl.ds(base + g * _CHUNK, _CHUNK)])

    return _sc_gather


def kernel(x_embed, prompt, prompt_key):
    similarity, idx = _sim_topk(x_embed, prompt_key)

    ld = (jnp.arange(L)[:, None, None, None] * DUAL
          + jnp.arange(DUAL)[None, None, :, None])
    src = ld * POOL + idx[None, :, None, :]
    src = src.reshape(_NW, _NCHUNK, _CHUNK).astype(jnp.int32)

    table = prompt.reshape(N_ROWS_TABLE, ROW)
    gathered = _sc_gather_fn()(table, src)
    batched_prompt = gathered.reshape(L, B, DUAL, TOPK * 20, 12, 64)
    return (batched_prompt, similarity, idx)


# speedup vs baseline: 2.5843x; 2.5843x over previous
import functools

import jax
import jax.numpy as jnp
from jax import lax
from jax.experimental import pallas as pl
from jax.experimental.pallas import tpu as pltpu
from jax.experimental.pallas import tpu_sc as plsc

B = 64
N_TOK = 196
D = 768
POOL = 512
TOPK = 4
L = 5
DUAL = 2
ROW = 20 * 12 * 64
N_ROWS_OUT = L * B * DUAL * TOPK
N_ROWS_TABLE = L * DUAL * POOL

B_BLK = 8
P_BLK = 128


def _sim_topk_body(x_ref, pk_ref, sim_ref, idx_ref):
    x_mean = jnp.mean(x_ref[...], axis=0)
    x_norm = x_mean * lax.rsqrt(
        jnp.maximum(jnp.sum(x_mean * x_mean, axis=-1, keepdims=True), 1e-12))
    pk = pk_ref[...]
    pk_norm = pk * lax.rsqrt(
        jnp.maximum(jnp.sum(pk * pk, axis=-1, keepdims=True), 1e-12))
    sim = jnp.dot(x_norm, pk_norm.T,
                  preferred_element_type=jnp.float32)
    sim_ref[...] = sim

    iota = lax.broadcasted_iota(jnp.int32, (B_BLK, POOL), 1)
    cur = sim
    cols = []
    for _ in range(TOPK):
        m = jnp.max(cur, axis=1, keepdims=True)
        j = jnp.min(jnp.where(cur == m, iota, POOL), axis=1)
        cols.append(j[:, None])
        cur = jnp.where(iota == j[:, None], jnp.float32(-jnp.inf), cur)
    idx_ref[...] = jnp.concatenate(cols, axis=1)


def _sim_topk(xv, prompt_key):
    return pl.pallas_call(
        _sim_topk_body,
        grid=(B // B_BLK,),
        in_specs=[
            pl.BlockSpec((N_TOK, B_BLK, D), lambda i: (0, i, 0)),
            pl.BlockSpec((POOL, D), lambda i: (0, 0)),
        ],
        out_specs=[
            pl.BlockSpec((B_BLK, POOL), lambda i: (i, 0)),
            pl.BlockSpec((B_BLK, TOPK), lambda i: (i, 0)),
        ],
        out_shape=[
            jax.ShapeDtypeStruct((B, POOL), jnp.float32),
            jax.ShapeDtypeStruct((B, TOPK), jnp.int32),
        ],
    )(xv, prompt_key)


def _transpose_body(x_ref, o_ref):
    x = x_ref[0, 0].reshape(ROW, P_BLK)
    o_ref[0, 0] = x.T.reshape(P_BLK, ROW)


def _pool_major_table(ptv):
    return pl.pallas_call(
        _transpose_body,
        grid=(L, DUAL, POOL // P_BLK),
        in_specs=[
            pl.BlockSpec((1, 1, 20, 768, P_BLK), lambda l, d, j: (l, d, 0, 0, j)),
        ],
        out_specs=pl.BlockSpec((1, 1, P_BLK, ROW), lambda l, d, j: (l, d, j, 0)),
        out_shape=jax.ShapeDtypeStruct((L, DUAL, POOL, ROW), jnp.float32),
    )(ptv)


_NW = 32
_RPW = N_ROWS_OUT // _NW
_CHUNK = 8
_NCHUNK = _RPW // _CHUNK


@functools.cache
def _sc_gather_fn():
    @functools.partial(
        pl.kernel,
        out_type=jax.ShapeDtypeStruct((N_ROWS_OUT, ROW), jnp.float32),
        mesh=plsc.VectorSubcoreMesh(core_axis_name="c", subcore_axis_name="s"),
        scratch_types=[
            pltpu.VMEM((_NCHUNK, _CHUNK), jnp.int32),
            pltpu.VMEM((_CHUNK, ROW), jnp.float32),
            pltpu.SemaphoreType.DMA,
        ],
    )
    def _sc_gather(table_hbm, src_hbm, out_hbm, idx_v, buf, sem):
        wid = lax.axis_index("s") * 2 + lax.axis_index("c")
        pltpu.sync_copy(src_hbm.at[wid], idx_v)
        base = wid * _RPW
        for g in range(_NCHUNK):
            pltpu.async_copy(table_hbm.at[idx_v.at[g]], buf, sem).wait()
            pltpu.sync_copy(buf, out_hbm.at[pl.ds(base + g * _CHUNK, _CHUNK)])

    return _sc_gather


def kernel(x_embed, prompt, prompt_key):
    xv = jnp.transpose(x_embed, (1, 0, 2))
    similarity, idx = _sim_topk(xv, prompt_key)

    ptv = jnp.transpose(prompt, (0, 1, 3, 4, 5, 2)).reshape(L, DUAL, 20, 768, POOL)
    table = _pool_major_table(ptv).reshape(N_ROWS_TABLE, ROW)

    ld = (jnp.arange(L)[:, None, None, None] * DUAL
          + jnp.arange(DUAL)[None, None, :, None])
    src = ld * POOL + idx[None, :, None, :]
    src = src.reshape(_NW, _NCHUNK, _CHUNK).astype(jnp.int32)

    gathered = _sc_gather_fn()(table, src)
    batched_prompt = gathered.reshape(L, B, DUAL, TOPK * 20, 12, 64)
    return (batched_prompt, similarity, idx)
